# baseline (speedup 1.0000x reference)
import jax
import jax.numpy as jnp
from jax.experimental import pallas as pl
from jax.experimental.pallas import tpu as pltpu

BS = 128
VOCAB = 100000
CBLK = 6400
NBLK = (VOCAB + CBLK - 1) // CBLK

NEG_BIG = -3.0e38
IDX_BIG = 2 ** 30


def _tc_body(l_ref, t_ref, p_ref, o_ref, mval, midx):
    k = pl.program_id(0)
    l = l_ref[...]
    t = t_ref[...]
    pen = p_ref[...]
    rp = 1.0 / pen
    f = jnp.where(l > 0.0, rp, pen)
    p = jnp.where(t > 0, l * f, l)
    gidx = k * CBLK + jax.lax.broadcasted_iota(jnp.int32, (BS, CBLK), 1)
    p = jnp.where(gidx < VOCAB, p, NEG_BIG)
    bmax = jnp.max(p, axis=1, keepdims=True)
    cand = jnp.where(p == bmax, gidx, IDX_BIG)
    barg = jnp.min(cand, axis=1, keepdims=True)

    @pl.when(k == 0)
    def _():
        mval[...] = bmax
        midx[...] = barg

    @pl.when(k > 0)
    def _():
        better = bmax > mval[...]
        mval[...] = jnp.where(better, bmax, mval[...])
        midx[...] = jnp.where(better, barg, midx[...])

    @pl.when(k == NBLK - 1)
    def _():
        o_ref[...] = midx[...]


def kernel(logits, repetition_penalty, token_count):
    l = logits.reshape(BS, VOCAB)
    out = pl.pallas_call(
        _tc_body,
        grid=(NBLK,),
        in_specs=[
            pl.BlockSpec((BS, CBLK), lambda k: (0, k)),
            pl.BlockSpec((BS, CBLK), lambda k: (0, k)),
            pl.BlockSpec((BS, 1), lambda k: (0, 0)),
        ],
        out_specs=pl.BlockSpec((BS, 1), lambda k: (0, 0)),
        out_shape=jax.ShapeDtypeStruct((BS, 1), jnp.int32),
        scratch_shapes=[
            pltpu.VMEM((BS, 1), jnp.float32),
            pltpu.VMEM((BS, 1), jnp.int32),
        ],
    )(l, token_count, repetition_penalty)
    return out
